# baseline (speedup 1.0000x reference)
import functools

import jax
import jax.numpy as jnp
from jax import lax
from jax.experimental import pallas as pl
from jax.experimental.pallas import tpu as pltpu
from jax.experimental.pallas import tpu_sc as plsc

N = 10000
E = 320000
D = 128

NC = 2
NS = 16
NW = NC * NS
EPW = E // NW
CH = 80
SLOTS = EPW // CH
NBUF = 3
RPT = 632

_MESH = plsc.VectorSubcoreMesh(core_axis_name="c", subcore_axis_name="s")


@functools.partial(
    pl.kernel,
    out_type=jax.ShapeDtypeStruct((NW * N,), jnp.float32),
    mesh=_MESH,
    compiler_params=pltpu.CompilerParams(needs_layout_passes=False),
    scratch_types=[
        pltpu.VMEM((EPW,), jnp.int32),
        pltpu.VMEM((EPW,), jnp.float32),
        pltpu.VMEM((N,), jnp.float32),
    ],
)
def _deg_kernel(dst_hbm, ew_hbm, out_hbm, dstv, ewv, degv):
    c = lax.axis_index("c")
    s = lax.axis_index("s")
    wid = s * NC + c
    base = wid * EPW
    pltpu.sync_copy(dst_hbm.at[pl.ds(base, EPW)], dstv)
    pltpu.sync_copy(ew_hbm.at[pl.ds(base, EPW)], ewv)
    zeros = jnp.zeros((16,), jnp.float32)

    def zbody(i, _):
        degv[pl.ds(i * 16, 16)] = zeros
        return 0

    lax.fori_loop(0, N // 16, zbody, 0)

    def ebody(i, _):
        idx = dstv[pl.ds(i * 16, 16)]
        w = ewv[pl.ds(i * 16, 16)]
        plsc.addupdate_scatter(degv, [idx], w)
        return 0

    lax.fori_loop(0, EPW // 16, ebody, 0)
    pltpu.sync_copy(degv, out_hbm.at[pl.ds(pl.multiple_of(wid * N, 8), N)])


@functools.partial(
    pl.kernel,
    out_type=jax.ShapeDtypeStruct((NC, N, D), jnp.float32),
    mesh=_MESH,
    compiler_params=pltpu.CompilerParams(needs_layout_passes=False),
    scratch_types=[
        pltpu.VMEM((NBUF, CH), jnp.int32),
        pltpu.VMEM((NBUF, CH), jnp.int32),
        pltpu.VMEM((NBUF, CH), jnp.float32),
        pltpu.VMEM((NBUF, CH, D), jnp.float32),
        pltpu.VMEM_SHARED((N, D), jnp.float32),
    ] + [pltpu.SemaphoreType.DMA] * (3 * NBUF),
)
def _agg_kernel(g_hbm, src_hbm, dst_hbm, ew_hbm, out_hbm,
                srcr, dstr, ewr, rows, acc, *sems):
    csem = sems[:NBUF]
    gsem = sems[NBUF:2 * NBUF]
    ssem = sems[2 * NBUF:]
    c = lax.axis_index("c")
    s = lax.axis_index("s")
    wid = s * NC + c
    ebase = wid * EPW
    zeros = jnp.zeros((16,), jnp.float32)

    rows0 = rows.at[0]

    def zbody(i, _):
        for k in range(8):
            rows0[i, pl.ds(k * 16, 16)] = zeros
        return 0

    lax.fori_loop(0, CH, zbody, 0)
    row0 = pl.multiple_of(
        jnp.where(s == NS - 1, N - RPT, s * RPT).astype(jnp.int32), 8)
    for k in range(RPT // CH):
        pltpu.sync_copy(rows0, acc.at[pl.ds(row0 + k * CH, CH)])
    pltpu.sync_copy(rows0.at[pl.ds(0, RPT % CH)],
                    acc.at[pl.ds(row0 + (RPT // CH) * CH, RPT % CH)])

    def copies(t, b):
        sl = pl.ds(ebase + t * CH, CH)
        pltpu.async_copy(src_hbm.at[sl], srcr.at[b], csem[b])
        pltpu.async_copy(dst_hbm.at[sl], dstr.at[b], csem[b])
        pltpu.async_copy(ew_hbm.at[sl], ewr.at[b], csem[b])

    def wait_copies(b):
        z = pl.ds(0, CH)
        pltpu.make_async_copy(src_hbm.at[z], srcr.at[b], csem[b]).wait()
        pltpu.make_async_copy(dst_hbm.at[z], dstr.at[b], csem[b]).wait()
        pltpu.make_async_copy(ew_hbm.at[z], ewr.at[b], csem[b]).wait()

    def gather(t, b):
        pltpu.async_copy(g_hbm.at[srcr.at[b]], rows.at[b], gsem[b])

    copies(0, 0)
    copies(1, 1)
    wait_copies(0)
    gather(0, 0)
    plsc.subcore_barrier()

    def scale_rows(rref, b):
        bvec = jnp.full((16,), b, jnp.int32)

        def rbody(j, _):
            w = plsc.load_gather(ewr, [bvec, jnp.full((16,), j, jnp.int32)])
            for k in range(8):
                sl = pl.ds(k * 16, 16)
                rref[j, sl] = rref[j, sl] * w
            return 0

        lax.fori_loop(0, CH, rbody, 0)

    def slot(t, b):
        rb = rows.at[b]
        bp = (b + NBUF - 1) % NBUF
        b1 = (b + 1) % NBUF
        pltpu.make_async_copy(g_hbm.at[pl.ds(0, CH)], rb, gsem[b]).wait()
        scale_rows(rb, b)
        pltpu.async_copy(rb, acc.at[dstr.at[b]], ssem[b], add=True)

        @pl.when(t >= 1)
        def _():
            pltpu.make_async_copy(g_hbm.at[pl.ds(0, CH)], rows.at[bp],
                                  ssem[bp]).wait()

        @pl.when(t + 2 < SLOTS)
        def _():
            copies(t + 2, bp)

        @pl.when(t + 1 < SLOTS)
        def _():
            wait_copies(b1)
            gather(t + 1, b1)

    def outer(gi, _):
        for b in range(NBUF):
            slot(gi * NBUF + b, b)
        return 0

    nfull = SLOTS // NBUF
    lax.fori_loop(0, nfull, outer, 0)
    for tt in range(nfull * NBUF, SLOTS):
        slot(jnp.int32(tt), tt % NBUF)

    bl = (SLOTS - 1) % NBUF
    pltpu.make_async_copy(g_hbm.at[pl.ds(0, CH)], rows.at[bl], ssem[bl]).wait()

    plsc.subcore_barrier()
    pltpu.sync_copy(acc.at[pl.ds(row0, RPT)], out_hbm.at[c, pl.ds(row0, RPT)])


_BM = 1000
_GRID = N // _BM


def _mm1_body(x_ref, degp_ref, w_ref, g_ref, dis_ref):
    deg = jnp.sum(degp_ref[...], axis=1) + 1.0
    dis = jnp.where(deg > 0, lax.rsqrt(deg), 0.0)
    hw = jnp.dot(x_ref[...], w_ref[...], preferred_element_type=jnp.float32)
    g_ref[...] = hw * dis[:, None]
    dis_ref[...] = dis[:, None]


def _mm_mid_body(p0_ref, p1_ref, g_ref, dis_ref, b_ref, m_ref, w_ref, go_ref):
    dis = dis_ref[...]
    u = b_ref[...] + dis * (p0_ref[...] + p1_ref[...] + g_ref[...])
    z = m_ref[...] * jnp.where(u > 0, u, (jnp.exp(u) - 1.0))
    go_ref[...] = jnp.dot(z, w_ref[...],
                          preferred_element_type=jnp.float32) * dis


def _mm_fin_body(p0_ref, p1_ref, g_ref, dis_ref, b_ref, m_ref, w_ref, bl_ref,
                 o_ref):
    dis = dis_ref[...]
    u = b_ref[...] + dis * (p0_ref[...] + p1_ref[...] + g_ref[...])
    z = m_ref[...] * jnp.where(u > 0, u, (jnp.exp(u) - 1.0))
    o_ref[...] = jnp.dot(z, w_ref[...],
                         preferred_element_type=jnp.float32) + bl_ref[...]


def _row_spec(d):
    return pl.BlockSpec((_BM, d), lambda i: (i, 0))


def _full_spec(r, cdim):
    return pl.BlockSpec((r, cdim), lambda i: (0, 0))


def _mm1(x, degp, w1):
    return pl.pallas_call(
        _mm1_body,
        grid=(_GRID,),
        in_specs=[_row_spec(D),
                  pl.BlockSpec((_BM, NW), lambda i: (i, 0)),
                  _full_spec(D, D)],
        out_specs=[_row_spec(D), _row_spec(1)],
        out_shape=[jax.ShapeDtypeStruct((N, D), jnp.float32),
                   jax.ShapeDtypeStruct((N, 1), jnp.float32)],
    )(x, degp, w1)


def _mm_mid(p0, p1, g, dis, b, m, w):
    return pl.pallas_call(
        _mm_mid_body,
        grid=(_GRID,),
        in_specs=[_row_spec(D), _row_spec(D), _row_spec(D), _row_spec(1),
                  _full_spec(1, D), _row_spec(D), _full_spec(D, D)],
        out_specs=_row_spec(D),
        out_shape=jax.ShapeDtypeStruct((N, D), jnp.float32),
    )(p0, p1, g, dis, b, m, w)


def _mm_fin(p0, p1, g, dis, b, m, wl, bl):
    return pl.pallas_call(
        _mm_fin_body,
        grid=(_GRID,),
        in_specs=[_row_spec(D), _row_spec(D), _row_spec(D), _row_spec(1),
                  _full_spec(1, D), _row_spec(D), _full_spec(D, 1),
                  _full_spec(1, 1)],
        out_specs=_row_spec(1),
        out_shape=jax.ShapeDtypeStruct((N, 1), jnp.float32),
    )(p0, p1, g, dis, b, m, wl, bl)


def kernel(x, edge_index, edge_attr, W1, b1, W2, b2, W3, b3, Wl, bl):
    src = edge_index[0]
    dst = edge_index[1]
    ew = edge_attr

    dkey = jax.random.key(42)
    masks = [
        jax.random.bernoulli(jax.random.fold_in(dkey, i), 0.5,
                             (N, D)).astype(jnp.float32) * 2.0
        for i in range(3)
    ]

    degp = _deg_kernel(dst, ew).reshape(NW, N).T
    g1, dis = _mm1(x, degp, W1)
    p = _agg_kernel(g1, src, dst, ew)
    g2 = _mm_mid(p[0], p[1], g1, dis, b1.reshape(1, D), masks[0], W2)
    p = _agg_kernel(g2, src, dst, ew)
    g3 = _mm_mid(p[0], p[1], g2, dis, b2.reshape(1, D), masks[1], W3)
    p = _agg_kernel(g3, src, dst, ew)
    out = _mm_fin(p[0], p[1], g3, dis, b3.reshape(1, D), masks[2], Wl,
                  bl.reshape(1, 1))
    return out


# speedup vs baseline: 1.8150x; 1.8150x over previous
import functools

import jax
import jax.numpy as jnp
from jax import lax
from jax.experimental import pallas as pl
from jax.experimental.pallas import tpu as pltpu
from jax.experimental.pallas import tpu_sc as plsc

N = 10000
E = 320000
D = 128

NC = 2
NS = 16
NW = NC * NS
EPW = E // NW
CH = 40
SLOTS = EPW // CH
NBUF = 7
GL = 4
CL = NBUF - 1
RPT = 632

_MESH = plsc.VectorSubcoreMesh(core_axis_name="c", subcore_axis_name="s")


@functools.partial(
    pl.kernel,
    out_type=jax.ShapeDtypeStruct((NW * N,), jnp.float32),
    mesh=_MESH,
    compiler_params=pltpu.CompilerParams(needs_layout_passes=False),
    scratch_types=[
        pltpu.VMEM((EPW,), jnp.int32),
        pltpu.VMEM((EPW,), jnp.float32),
        pltpu.VMEM((N,), jnp.float32),
    ],
)
def _deg_kernel(dst_hbm, ew_hbm, out_hbm, dstv, ewv, degv):
    c = lax.axis_index("c")
    s = lax.axis_index("s")
    wid = s * NC + c
    base = wid * EPW
    pltpu.sync_copy(dst_hbm.at[pl.ds(base, EPW)], dstv)
    pltpu.sync_copy(ew_hbm.at[pl.ds(base, EPW)], ewv)
    zeros = jnp.zeros((16,), jnp.float32)

    def zbody(i, _):
        degv[pl.ds(i * 16, 16)] = zeros
        return 0

    lax.fori_loop(0, N // 16, zbody, 0)

    def ebody(i, _):
        idx = dstv[pl.ds(i * 16, 16)]
        w = ewv[pl.ds(i * 16, 16)]
        plsc.addupdate_scatter(degv, [idx], w)
        return 0

    lax.fori_loop(0, EPW // 16, ebody, 0)
    pltpu.sync_copy(degv, out_hbm.at[pl.ds(pl.multiple_of(wid * N, 8), N)])


@functools.partial(
    pl.kernel,
    out_type=jax.ShapeDtypeStruct((NC, N, D), jnp.float32),
    mesh=_MESH,
    compiler_params=pltpu.CompilerParams(needs_layout_passes=False),
    scratch_types=[
        pltpu.VMEM((NBUF, CH), jnp.int32),
        pltpu.VMEM((NBUF, CH), jnp.int32),
        pltpu.VMEM((NBUF, CH), jnp.float32),
        pltpu.VMEM((NBUF, CH, D), jnp.float32),
        pltpu.VMEM_SHARED((N, D), jnp.float32),
    ] + [pltpu.SemaphoreType.DMA] * (3 * NBUF),
)
def _agg_kernel(g_hbm, src_hbm, dst_hbm, ew_hbm, out_hbm,
                srcr, dstr, ewr, rows, acc, *sems):
    csem = sems[:NBUF]
    gsem = sems[NBUF:2 * NBUF]
    ssem = sems[2 * NBUF:]
    c = lax.axis_index("c")
    s = lax.axis_index("s")
    wid = s * NC + c
    ebase = wid * EPW
    zeros = jnp.zeros((16,), jnp.float32)

    rows0 = rows.at[0]

    def zbody(i, _):
        for k in range(8):
            rows0[i, pl.ds(k * 16, 16)] = zeros
        return 0

    lax.fori_loop(0, CH, zbody, 0)
    row0 = pl.multiple_of(
        jnp.where(s == NS - 1, N - RPT, s * RPT).astype(jnp.int32), 8)
    for k in range(RPT // CH):
        pltpu.sync_copy(rows0, acc.at[pl.ds(row0 + k * CH, CH)])
    pltpu.sync_copy(rows0.at[pl.ds(0, RPT % CH)],
                    acc.at[pl.ds(row0 + (RPT // CH) * CH, RPT % CH)])

    def copies(t, b):
        sl = pl.ds(ebase + t * CH, CH)
        pltpu.async_copy(src_hbm.at[sl], srcr.at[b], csem[b])
        pltpu.async_copy(dst_hbm.at[sl], dstr.at[b], csem[b])
        pltpu.async_copy(ew_hbm.at[sl], ewr.at[b], csem[b])

    def wait_copies(b):
        z = pl.ds(0, CH)
        pltpu.make_async_copy(src_hbm.at[z], srcr.at[b], csem[b]).wait()
        pltpu.make_async_copy(dst_hbm.at[z], dstr.at[b], csem[b]).wait()
        pltpu.make_async_copy(ew_hbm.at[z], ewr.at[b], csem[b]).wait()

    def gather(t, b):
        pltpu.async_copy(g_hbm.at[srcr.at[b]], rows.at[b], gsem[b])

    for b in range(CL):
        copies(b, b)
    for b in range(GL):
        wait_copies(b)
        gather(b, b)
    plsc.subcore_barrier()

    def scale_rows(rref, b):
        bvec = jnp.full((16,), b, jnp.int32)

        def rbody(j, _):
            w = plsc.load_gather(ewr, [bvec, jnp.full((16,), j, jnp.int32)])
            for k in range(8):
                sl = pl.ds(k * 16, 16)
                rref[j, sl] = rref[j, sl] * w
            return 0

        lax.fori_loop(0, CH, rbody, 0)

    def slot(t, b):
        rb = rows.at[b]
        bp = (b + NBUF - 1) % NBUF
        pltpu.make_async_copy(g_hbm.at[pl.ds(0, CH)], rb, gsem[b]).wait()
        scale_rows(rb, b)
        pltpu.async_copy(rb, acc.at[dstr.at[b]], ssem[b], add=True)

        @pl.when(t >= 1)
        def _():
            pltpu.make_async_copy(g_hbm.at[pl.ds(0, CH)], rows.at[bp],
                                  ssem[bp]).wait()

        @pl.when(t + CL < SLOTS)
        def _():
            copies(t + CL, bp)

        @pl.when(t + GL < SLOTS)
        def _():
            bg = (b + GL) % NBUF
            wait_copies(bg)
            gather(t + GL, bg)

    def outer(gi, _):
        for b in range(NBUF):
            slot(gi * NBUF + b, b)
        return 0

    nfull = SLOTS // NBUF
    lax.fori_loop(0, nfull, outer, 0)
    for tt in range(nfull * NBUF, SLOTS):
        slot(jnp.int32(tt), tt % NBUF)

    bl = (SLOTS - 1) % NBUF
    pltpu.make_async_copy(g_hbm.at[pl.ds(0, CH)], rows.at[bl], ssem[bl]).wait()

    plsc.subcore_barrier()
    pltpu.sync_copy(acc.at[pl.ds(row0, RPT)], out_hbm.at[c, pl.ds(row0, RPT)])


_BM = 1000
_GRID = N // _BM


def _mm1_body(x_ref, degp_ref, w_ref, g_ref, dis_ref):
    deg = jnp.sum(degp_ref[...], axis=1) + 1.0
    dis = jnp.where(deg > 0, lax.rsqrt(deg), 0.0)
    hw = jnp.dot(x_ref[...], w_ref[...], preferred_element_type=jnp.float32)
    g_ref[...] = hw * dis[:, None]
    dis_ref[...] = dis[:, None]


def _mm_mid_body(p0_ref, p1_ref, g_ref, dis_ref, b_ref, m_ref, w_ref, go_ref):
    dis = dis_ref[...]
    u = b_ref[...] + dis * (p0_ref[...] + p1_ref[...] + g_ref[...])
    z = m_ref[...] * jnp.where(u > 0, u, (jnp.exp(u) - 1.0))
    go_ref[...] = jnp.dot(z, w_ref[...],
                          preferred_element_type=jnp.float32) * dis


def _mm_fin_body(p0_ref, p1_ref, g_ref, dis_ref, b_ref, m_ref, w_ref, bl_ref,
                 o_ref):
    dis = dis_ref[...]
    u = b_ref[...] + dis * (p0_ref[...] + p1_ref[...] + g_ref[...])
    z = m_ref[...] * jnp.where(u > 0, u, (jnp.exp(u) - 1.0))
    o_ref[...] = jnp.dot(z, w_ref[...],
                         preferred_element_type=jnp.float32) + bl_ref[...]


def _row_spec(d):
    return pl.BlockSpec((_BM, d), lambda i: (i, 0))


def _full_spec(r, cdim):
    return pl.BlockSpec((r, cdim), lambda i: (0, 0))


def _mm1(x, degp, w1):
    return pl.pallas_call(
        _mm1_body,
        grid=(_GRID,),
        in_specs=[_row_spec(D),
                  pl.BlockSpec((_BM, NW), lambda i: (i, 0)),
                  _full_spec(D, D)],
        out_specs=[_row_spec(D), _row_spec(1)],
        out_shape=[jax.ShapeDtypeStruct((N, D), jnp.float32),
                   jax.ShapeDtypeStruct((N, 1), jnp.float32)],
    )(x, degp, w1)


def _mm_mid(p0, p1, g, dis, b, m, w):
    return pl.pallas_call(
        _mm_mid_body,
        grid=(_GRID,),
        in_specs=[_row_spec(D), _row_spec(D), _row_spec(D), _row_spec(1),
                  _full_spec(1, D), _row_spec(D), _full_spec(D, D)],
        out_specs=_row_spec(D),
        out_shape=jax.ShapeDtypeStruct((N, D), jnp.float32),
    )(p0, p1, g, dis, b, m, w)


def _mm_fin(p0, p1, g, dis, b, m, wl, bl):
    return pl.pallas_call(
        _mm_fin_body,
        grid=(_GRID,),
        in_specs=[_row_spec(D), _row_spec(D), _row_spec(D), _row_spec(1),
                  _full_spec(1, D), _row_spec(D), _full_spec(D, 1),
                  _full_spec(1, 1)],
        out_specs=_row_spec(1),
        out_shape=jax.ShapeDtypeStruct((N, 1), jnp.float32),
    )(p0, p1, g, dis, b, m, wl, bl)


def kernel(x, edge_index, edge_attr, W1, b1, W2, b2, W3, b3, Wl, bl):
    src = edge_index[0]
    dst = edge_index[1]
    ew = edge_attr

    dkey = jax.random.key(42)
    masks = [
        jax.random.bernoulli(jax.random.fold_in(dkey, i), 0.5,
                             (N, D)).astype(jnp.float32) * 2.0
        for i in range(3)
    ]

    degp = _deg_kernel(dst, ew).reshape(NW, N).T
    g1, dis = _mm1(x, degp, W1)
    p = _agg_kernel(g1, src, dst, ew)
    g2 = _mm_mid(p[0], p[1], g1, dis, b1.reshape(1, D), masks[0], W2)
    p = _agg_kernel(g2, src, dst, ew)
    g3 = _mm_mid(p[0], p[1], g2, dis, b2.reshape(1, D), masks[1], W3)
    p = _agg_kernel(g3, src, dst, ew)
    out = _mm_fin(p[0], p[1], g3, dis, b3.reshape(1, D), masks[2], Wl,
                  bl.reshape(1, 1))
    return out


# speedup vs baseline: 1.8179x; 1.0016x over previous
import functools

import jax
import jax.numpy as jnp
from jax import lax
from jax.experimental import pallas as pl
from jax.experimental.pallas import tpu as pltpu
from jax.experimental.pallas import tpu_sc as plsc

N = 10000
E = 320000
D = 128

NC = 2
NS = 16
NW = NC * NS
EPW = E // NW
CH = 40
SLOTS = EPW // CH
NBUF = 8
GL = 5
CL = NBUF - 1
RPT = 632

_MESH = plsc.VectorSubcoreMesh(core_axis_name="c", subcore_axis_name="s")


@functools.partial(
    pl.kernel,
    out_type=jax.ShapeDtypeStruct((NW * N,), jnp.float32),
    mesh=_MESH,
    compiler_params=pltpu.CompilerParams(needs_layout_passes=False),
    scratch_types=[
        pltpu.VMEM((EPW,), jnp.int32),
        pltpu.VMEM((EPW,), jnp.float32),
        pltpu.VMEM((N,), jnp.float32),
    ],
)
def _deg_kernel(dst_hbm, ew_hbm, out_hbm, dstv, ewv, degv):
    c = lax.axis_index("c")
    s = lax.axis_index("s")
    wid = s * NC + c
    base = wid * EPW
    pltpu.sync_copy(dst_hbm.at[pl.ds(base, EPW)], dstv)
    pltpu.sync_copy(ew_hbm.at[pl.ds(base, EPW)], ewv)
    zeros = jnp.zeros((16,), jnp.float32)

    def zbody(i, _):
        degv[pl.ds(i * 16, 16)] = zeros
        return 0

    lax.fori_loop(0, N // 16, zbody, 0)

    def ebody(i, _):
        idx = dstv[pl.ds(i * 16, 16)]
        w = ewv[pl.ds(i * 16, 16)]
        plsc.addupdate_scatter(degv, [idx], w)
        return 0

    lax.fori_loop(0, EPW // 16, ebody, 0)
    pltpu.sync_copy(degv, out_hbm.at[pl.ds(pl.multiple_of(wid * N, 8), N)])


@functools.partial(
    pl.kernel,
    out_type=jax.ShapeDtypeStruct((NC, N, D), jnp.float32),
    mesh=_MESH,
    compiler_params=pltpu.CompilerParams(needs_layout_passes=False),
    scratch_types=[
        pltpu.VMEM((NBUF, CH), jnp.int32),
        pltpu.VMEM((NBUF, CH), jnp.int32),
        pltpu.VMEM((NBUF, CH), jnp.float32),
        pltpu.VMEM((NBUF, CH, D), jnp.float32),
        pltpu.VMEM_SHARED((N, D), jnp.float32),
    ] + [pltpu.SemaphoreType.DMA] * (3 * NBUF),
)
def _agg_kernel(g_hbm, src_hbm, dst_hbm, ew_hbm, out_hbm,
                srcr, dstr, ewr, rows, acc, *sems):
    csem = sems[:NBUF]
    gsem = sems[NBUF:2 * NBUF]
    ssem = sems[2 * NBUF:]
    c = lax.axis_index("c")
    s = lax.axis_index("s")
    wid = s * NC + c
    ebase = wid * EPW
    zeros = jnp.zeros((16,), jnp.float32)

    rows0 = rows.at[0]

    def zbody(i, _):
        for k in range(8):
            rows0[i, pl.ds(k * 16, 16)] = zeros
        return 0

    lax.fori_loop(0, CH, zbody, 0)
    row0 = pl.multiple_of(
        jnp.where(s == NS - 1, N - RPT, s * RPT).astype(jnp.int32), 8)
    for k in range(RPT // CH):
        pltpu.sync_copy(rows0, acc.at[pl.ds(row0 + k * CH, CH)])
    pltpu.sync_copy(rows0.at[pl.ds(0, RPT % CH)],
                    acc.at[pl.ds(row0 + (RPT // CH) * CH, RPT % CH)])

    def copies(t, b):
        sl = pl.ds(ebase + t * CH, CH)
        pltpu.async_copy(src_hbm.at[sl], srcr.at[b], csem[b])
        pltpu.async_copy(dst_hbm.at[sl], dstr.at[b], csem[b])
        pltpu.async_copy(ew_hbm.at[sl], ewr.at[b], csem[b])

    def wait_copies(b):
        z = pl.ds(0, CH)
        pltpu.make_async_copy(src_hbm.at[z], srcr.at[b], csem[b]).wait()
        pltpu.make_async_copy(dst_hbm.at[z], dstr.at[b], csem[b]).wait()
        pltpu.make_async_copy(ew_hbm.at[z], ewr.at[b], csem[b]).wait()

    def gather(t, b):
        pltpu.async_copy(g_hbm.at[srcr.at[b]], rows.at[b], gsem[b])

    for b in range(CL):
        copies(b, b)
    for b in range(GL):
        wait_copies(b)
        gather(b, b)
    plsc.subcore_barrier()

    def scale_rows(rref, b):
        bvec = jnp.full((16,), b, jnp.int32)

        def rbody(j, _):
            w = plsc.load_gather(ewr, [bvec, jnp.full((16,), j, jnp.int32)])
            for k in range(8):
                sl = pl.ds(k * 16, 16)
                rref[j, sl] = rref[j, sl] * w
            return 0

        lax.fori_loop(0, CH, rbody, 0)

    def slot(t, b):
        rb = rows.at[b]
        bp = (b + NBUF - 1) % NBUF
        pltpu.make_async_copy(g_hbm.at[pl.ds(0, CH)], rb, gsem[b]).wait()
        scale_rows(rb, b)
        pltpu.async_copy(rb, acc.at[dstr.at[b]], ssem[b], add=True)

        @pl.when(t >= 1)
        def _():
            pltpu.make_async_copy(g_hbm.at[pl.ds(0, CH)], rows.at[bp],
                                  ssem[bp]).wait()

        @pl.when(t + CL < SLOTS)
        def _():
            copies(t + CL, bp)

        @pl.when(t + GL < SLOTS)
        def _():
            bg = (b + GL) % NBUF
            wait_copies(bg)
            gather(t + GL, bg)

    def outer(gi, _):
        for b in range(NBUF):
            slot(gi * NBUF + b, b)
        return 0

    nfull = SLOTS // NBUF
    lax.fori_loop(0, nfull, outer, 0)
    for tt in range(nfull * NBUF, SLOTS):
        slot(jnp.int32(tt), tt % NBUF)

    bl = (SLOTS - 1) % NBUF
    pltpu.make_async_copy(g_hbm.at[pl.ds(0, CH)], rows.at[bl], ssem[bl]).wait()

    plsc.subcore_barrier()
    pltpu.sync_copy(acc.at[pl.ds(row0, RPT)], out_hbm.at[c, pl.ds(row0, RPT)])


_BM = 1000
_GRID = N // _BM


def _mm1_body(x_ref, degp_ref, w_ref, g_ref, dis_ref):
    deg = jnp.sum(degp_ref[...], axis=1) + 1.0
    dis = jnp.where(deg > 0, lax.rsqrt(deg), 0.0)
    hw = jnp.dot(x_ref[...], w_ref[...], preferred_element_type=jnp.float32)
    g_ref[...] = hw * dis[:, None]
    dis_ref[...] = dis[:, None]


def _mm_mid_body(p0_ref, p1_ref, g_ref, dis_ref, b_ref, m_ref, w_ref, go_ref):
    dis = dis_ref[...]
    u = b_ref[...] + dis * (p0_ref[...] + p1_ref[...] + g_ref[...])
    z = m_ref[...] * jnp.where(u > 0, u, (jnp.exp(u) - 1.0))
    go_ref[...] = jnp.dot(z, w_ref[...],
                          preferred_element_type=jnp.float32) * dis


def _mm_fin_body(p0_ref, p1_ref, g_ref, dis_ref, b_ref, m_ref, w_ref, bl_ref,
                 o_ref):
    dis = dis_ref[...]
    u = b_ref[...] + dis * (p0_ref[...] + p1_ref[...] + g_ref[...])
    z = m_ref[...] * jnp.where(u > 0, u, (jnp.exp(u) - 1.0))
    o_ref[...] = jnp.dot(z, w_ref[...],
                         preferred_element_type=jnp.float32) + bl_ref[...]


def _row_spec(d):
    return pl.BlockSpec((_BM, d), lambda i: (i, 0))


def _full_spec(r, cdim):
    return pl.BlockSpec((r, cdim), lambda i: (0, 0))


def _mm1(x, degp, w1):
    return pl.pallas_call(
        _mm1_body,
        grid=(_GRID,),
        in_specs=[_row_spec(D),
                  pl.BlockSpec((_BM, NW), lambda i: (i, 0)),
                  _full_spec(D, D)],
        out_specs=[_row_spec(D), _row_spec(1)],
        out_shape=[jax.ShapeDtypeStruct((N, D), jnp.float32),
                   jax.ShapeDtypeStruct((N, 1), jnp.float32)],
    )(x, degp, w1)


def _mm_mid(p0, p1, g, dis, b, m, w):
    return pl.pallas_call(
        _mm_mid_body,
        grid=(_GRID,),
        in_specs=[_row_spec(D), _row_spec(D), _row_spec(D), _row_spec(1),
                  _full_spec(1, D), _row_spec(D), _full_spec(D, D)],
        out_specs=_row_spec(D),
        out_shape=jax.ShapeDtypeStruct((N, D), jnp.float32),
    )(p0, p1, g, dis, b, m, w)


def _mm_fin(p0, p1, g, dis, b, m, wl, bl):
    return pl.pallas_call(
        _mm_fin_body,
        grid=(_GRID,),
        in_specs=[_row_spec(D), _row_spec(D), _row_spec(D), _row_spec(1),
                  _full_spec(1, D), _row_spec(D), _full_spec(D, 1),
                  _full_spec(1, 1)],
        out_specs=_row_spec(1),
        out_shape=jax.ShapeDtypeStruct((N, 1), jnp.float32),
    )(p0, p1, g, dis, b, m, wl, bl)


def kernel(x, edge_index, edge_attr, W1, b1, W2, b2, W3, b3, Wl, bl):
    src = edge_index[0]
    dst = edge_index[1]
    ew = edge_attr

    dkey = jax.random.key(42)
    masks = [
        jax.random.bernoulli(jax.random.fold_in(dkey, i), 0.5,
                             (N, D)).astype(jnp.float32) * 2.0
        for i in range(3)
    ]

    degp = _deg_kernel(dst, ew).reshape(NW, N).T
    g1, dis = _mm1(x, degp, W1)
    p = _agg_kernel(g1, src, dst, ew)
    g2 = _mm_mid(p[0], p[1], g1, dis, b1.reshape(1, D), masks[0], W2)
    p = _agg_kernel(g2, src, dst, ew)
    g3 = _mm_mid(p[0], p[1], g2, dis, b2.reshape(1, D), masks[1], W3)
    p = _agg_kernel(g3, src, dst, ew)
    out = _mm_fin(p[0], p[1], g3, dis, b3.reshape(1, D), masks[2], Wl,
                  bl.reshape(1, 1))
    return out


# speedup vs baseline: 1.9020x; 1.0463x over previous
import functools

import jax
import jax.numpy as jnp
from jax import lax
from jax.experimental import pallas as pl
from jax.experimental.pallas import tpu as pltpu
from jax.experimental.pallas import tpu_sc as plsc

N = 10000
E = 320000
D = 128

NC = 2
NS = 16
NW = NC * NS
EPW = E // NW
CH = 40
SLOTS = EPW // CH
NBUF = 8
GL = 5
CL = NBUF - 1
RPT = 632

_MESH = plsc.VectorSubcoreMesh(core_axis_name="c", subcore_axis_name="s")


@functools.partial(
    pl.kernel,
    out_type=jax.ShapeDtypeStruct((NW * N,), jnp.float32),
    mesh=_MESH,
    compiler_params=pltpu.CompilerParams(needs_layout_passes=False),
    scratch_types=[
        pltpu.VMEM((EPW,), jnp.int32),
        pltpu.VMEM((EPW,), jnp.float32),
        pltpu.VMEM((N,), jnp.float32),
    ],
)
def _deg_kernel(dst_hbm, ew_hbm, out_hbm, dstv, ewv, degv):
    c = lax.axis_index("c")
    s = lax.axis_index("s")
    wid = s * NC + c
    base = wid * EPW
    pltpu.sync_copy(dst_hbm.at[pl.ds(base, EPW)], dstv)
    pltpu.sync_copy(ew_hbm.at[pl.ds(base, EPW)], ewv)
    zeros = jnp.zeros((16,), jnp.float32)

    def zbody(i, _):
        degv[pl.ds(i * 16, 16)] = zeros
        return 0

    lax.fori_loop(0, N // 16, zbody, 0)

    def ebody(i, _):
        idx = dstv[pl.ds(i * 16, 16)]
        w = ewv[pl.ds(i * 16, 16)]
        plsc.addupdate_scatter(degv, [idx], w)
        return 0

    lax.fori_loop(0, EPW // 16, ebody, 0)
    pltpu.sync_copy(degv, out_hbm.at[pl.ds(pl.multiple_of(wid * N, 8), N)])


@functools.partial(
    pl.kernel,
    out_type=jax.ShapeDtypeStruct((NC, N, D), jnp.float32),
    mesh=_MESH,
    compiler_params=pltpu.CompilerParams(needs_layout_passes=False),
    scratch_types=[
        pltpu.VMEM((NBUF, CH), jnp.int32),
        pltpu.VMEM((NBUF, CH), jnp.int32),
        pltpu.VMEM((NBUF, CH), jnp.float32),
        pltpu.VMEM((NBUF, CH, D), jnp.float32),
        pltpu.VMEM_SHARED((N, D), jnp.float32),
    ] + [pltpu.SemaphoreType.DMA] * (3 * NBUF),
)
def _agg_kernel(g_hbm, src_hbm, dst_hbm, ew_hbm, out_hbm,
                srcr, dstr, ewr, rows, acc, *sems):
    csem = sems[:NBUF]
    gsem = sems[NBUF:2 * NBUF]
    ssem = sems[2 * NBUF:]
    c = lax.axis_index("c")
    s = lax.axis_index("s")
    wid = s * NC + c
    ebase = wid * EPW
    zeros = jnp.zeros((16,), jnp.float32)

    rows0 = rows.at[0]

    def zbody(i, _):
        for k in range(8):
            rows0[i, pl.ds(k * 16, 16)] = zeros
        return 0

    lax.fori_loop(0, CH, zbody, 0)
    row0 = pl.multiple_of(
        jnp.where(s == NS - 1, N - RPT, s * RPT).astype(jnp.int32), 8)
    for k in range(RPT // CH):
        pltpu.sync_copy(rows0, acc.at[pl.ds(row0 + k * CH, CH)])
    pltpu.sync_copy(rows0.at[pl.ds(0, RPT % CH)],
                    acc.at[pl.ds(row0 + (RPT // CH) * CH, RPT % CH)])

    def copies(t, b):
        sl = pl.ds(ebase + t * CH, CH)
        pltpu.async_copy(src_hbm.at[sl], srcr.at[b], csem[b])
        pltpu.async_copy(dst_hbm.at[sl], dstr.at[b], csem[b])
        pltpu.async_copy(ew_hbm.at[sl], ewr.at[b], csem[b])

    def wait_copies(b):
        z = pl.ds(0, CH)
        pltpu.make_async_copy(src_hbm.at[z], srcr.at[b], csem[b]).wait()
        pltpu.make_async_copy(dst_hbm.at[z], dstr.at[b], csem[b]).wait()
        pltpu.make_async_copy(ew_hbm.at[z], ewr.at[b], csem[b]).wait()

    def gather(t, b):
        pltpu.async_copy(g_hbm.at[srcr.at[b]], rows.at[b], gsem[b])

    for b in range(CL):
        copies(b, b)
    for b in range(GL):
        wait_copies(b)
        gather(b, b)
    plsc.subcore_barrier()

    def scale_rows(rref, b):
        bvec = jnp.full((16,), b, jnp.int32)

        @plsc.parallel_loop(0, CH, 1, unroll=4)
        def _(j):
            w = plsc.load_gather(ewr, [bvec, jnp.full((16,), j, jnp.int32)])
            for k in range(8):
                sl = pl.ds(k * 16, 16)
                rref[j, sl] = rref[j, sl] * w

    def slot(t, b):
        rb = rows.at[b]
        bp = (b + NBUF - 1) % NBUF
        pltpu.make_async_copy(g_hbm.at[pl.ds(0, CH)], rb, gsem[b]).wait()
        scale_rows(rb, b)
        pltpu.async_copy(rb, acc.at[dstr.at[b]], ssem[b], add=True)

        @pl.when(t >= 1)
        def _():
            pltpu.make_async_copy(g_hbm.at[pl.ds(0, CH)], rows.at[bp],
                                  ssem[bp]).wait()

        @pl.when(t + CL < SLOTS)
        def _():
            copies(t + CL, bp)

        @pl.when(t + GL < SLOTS)
        def _():
            bg = (b + GL) % NBUF
            wait_copies(bg)
            gather(t + GL, bg)

    def outer(gi, _):
        for b in range(NBUF):
            slot(gi * NBUF + b, b)
        return 0

    nfull = SLOTS // NBUF
    lax.fori_loop(0, nfull, outer, 0)
    for tt in range(nfull * NBUF, SLOTS):
        slot(jnp.int32(tt), tt % NBUF)

    bl = (SLOTS - 1) % NBUF
    pltpu.make_async_copy(g_hbm.at[pl.ds(0, CH)], rows.at[bl], ssem[bl]).wait()

    plsc.subcore_barrier()
    pltpu.sync_copy(acc.at[pl.ds(row0, RPT)], out_hbm.at[c, pl.ds(row0, RPT)])


_BM = 1000
_GRID = N // _BM


def _mm1_body(x_ref, degp_ref, w_ref, g_ref, dis_ref):
    deg = jnp.sum(degp_ref[...], axis=1) + 1.0
    dis = jnp.where(deg > 0, lax.rsqrt(deg), 0.0)
    hw = jnp.dot(x_ref[...], w_ref[...], preferred_element_type=jnp.float32)
    g_ref[...] = hw * dis[:, None]
    dis_ref[...] = dis[:, None]


def _mm_mid_body(p0_ref, p1_ref, g_ref, dis_ref, b_ref, m_ref, w_ref, go_ref):
    dis = dis_ref[...]
    u = b_ref[...] + dis * (p0_ref[...] + p1_ref[...] + g_ref[...])
    z = m_ref[...] * jnp.where(u > 0, u, (jnp.exp(u) - 1.0))
    go_ref[...] = jnp.dot(z, w_ref[...],
                          preferred_element_type=jnp.float32) * dis


def _mm_fin_body(p0_ref, p1_ref, g_ref, dis_ref, b_ref, m_ref, w_ref, bl_ref,
                 o_ref):
    dis = dis_ref[...]
    u = b_ref[...] + dis * (p0_ref[...] + p1_ref[...] + g_ref[...])
    z = m_ref[...] * jnp.where(u > 0, u, (jnp.exp(u) - 1.0))
    o_ref[...] = jnp.dot(z, w_ref[...],
                         preferred_element_type=jnp.float32) + bl_ref[...]


def _row_spec(d):
    return pl.BlockSpec((_BM, d), lambda i: (i, 0))


def _full_spec(r, cdim):
    return pl.BlockSpec((r, cdim), lambda i: (0, 0))


def _mm1(x, degp, w1):
    return pl.pallas_call(
        _mm1_body,
        grid=(_GRID,),
        in_specs=[_row_spec(D),
                  pl.BlockSpec((_BM, NW), lambda i: (i, 0)),
                  _full_spec(D, D)],
        out_specs=[_row_spec(D), _row_spec(1)],
        out_shape=[jax.ShapeDtypeStruct((N, D), jnp.float32),
                   jax.ShapeDtypeStruct((N, 1), jnp.float32)],
    )(x, degp, w1)


def _mm_mid(p0, p1, g, dis, b, m, w):
    return pl.pallas_call(
        _mm_mid_body,
        grid=(_GRID,),
        in_specs=[_row_spec(D), _row_spec(D), _row_spec(D), _row_spec(1),
                  _full_spec(1, D), _row_spec(D), _full_spec(D, D)],
        out_specs=_row_spec(D),
        out_shape=jax.ShapeDtypeStruct((N, D), jnp.float32),
    )(p0, p1, g, dis, b, m, w)


def _mm_fin(p0, p1, g, dis, b, m, wl, bl):
    return pl.pallas_call(
        _mm_fin_body,
        grid=(_GRID,),
        in_specs=[_row_spec(D), _row_spec(D), _row_spec(D), _row_spec(1),
                  _full_spec(1, D), _row_spec(D), _full_spec(D, 1),
                  _full_spec(1, 1)],
        out_specs=_row_spec(1),
        out_shape=jax.ShapeDtypeStruct((N, 1), jnp.float32),
    )(p0, p1, g, dis, b, m, wl, bl)


def kernel(x, edge_index, edge_attr, W1, b1, W2, b2, W3, b3, Wl, bl):
    src = edge_index[0]
    dst = edge_index[1]
    ew = edge_attr

    dkey = jax.random.key(42)
    masks = [
        jax.random.bernoulli(jax.random.fold_in(dkey, i), 0.5,
                             (N, D)).astype(jnp.float32) * 2.0
        for i in range(3)
    ]

    degp = _deg_kernel(dst, ew).reshape(NW, N).T
    g1, dis = _mm1(x, degp, W1)
    p = _agg_kernel(g1, src, dst, ew)
    g2 = _mm_mid(p[0], p[1], g1, dis, b1.reshape(1, D), masks[0], W2)
    p = _agg_kernel(g2, src, dst, ew)
    g3 = _mm_mid(p[0], p[1], g2, dis, b2.reshape(1, D), masks[1], W3)
    p = _agg_kernel(g3, src, dst, ew)
    out = _mm_fin(p[0], p[1], g3, dis, b3.reshape(1, D), masks[2], Wl,
                  bl.reshape(1, 1))
    return out
